# baseline (speedup 1.0000x reference)
import jax
import jax.numpy as jnp
from jax import lax
from jax.experimental import pallas as pl
from jax.experimental.pallas import tpu as pltpu
from jax.experimental.pallas import tpu_sc as plsc

N_NODES = 10000
N_EDGES = 320000
D = 128
DH = D // 2
N_REL = 50

NC = 2
NS = 16
CHUNK = 128
CHUNKS_PER_W = 160
EPW = CHUNKS_PER_W * CHUNK
E_PAD = NS * EPW
N_PAD = 10016
ACC_ROWS = 10240
ROWS_PER_TILE = ACC_ROWS // NS


def _tc_prep(x_ref, wl_ref, wa_ref, rel_ref, xt_ref, s_ref, r_ref):
    xv = x_ref[...]
    xt = lax.dot_general(
        xv, wl_ref[...], (((1,), (1,)), ((), ())),
        preferred_element_type=jnp.float32)
    pad_z = jnp.zeros((N_PAD - N_NODES, DH), jnp.float32)
    xt_ref[0:N_NODES, :] = xt[:, 0:DH]
    xt_ref[N_NODES:N_PAD, :] = pad_z
    xt_ref[N_PAD:N_PAD + N_NODES, :] = xt[:, DH:D]
    xt_ref[N_PAD + N_NODES:2 * N_PAD, :] = pad_z
    wa = wa_ref[...]
    sv = lax.dot_general(
        xv, wa, (((1,), (1,)), ((), ())), preferred_element_type=jnp.float32)
    s_ref[0:N_NODES, :] = sv
    s_ref[N_NODES:N_PAD, :] = jnp.zeros((N_PAD - N_NODES, 1), jnp.float32)
    r_ref[...] = lax.dot_general(
        rel_ref[...], wa, (((1,), (1,)), ((), ())),
        preferred_element_type=jnp.float32)


def _sc_edges(xt_hbm, s_hbm, r_hbm, src_hbm, tgt_hbm, typ_hbm, out_hbm,
              src_v, tgt_v, s_v, r_v, typb,
              rows0, rows1, rows2, rows3, acc,
              gsem, gsemb, ssem, tsem):
    c = lax.axis_index("c")
    s = lax.axis_index("s")
    bufs = (rows0, rows1, rows2, rows3)

    @pl.loop(0, CHUNK)
    def _zero_rows(i):
        zero16 = jnp.zeros((16,), jnp.float32)
        for h in range(DH // 16):
            rows0[i, pl.ds(h * 16, 16)] = zero16

    for b in range(ROWS_PER_TILE // CHUNK):
        pltpu.sync_copy(rows0, acc.at[pl.ds((s * 5 + b) * CHUNK, CHUNK)])
    plsc.subcore_barrier()

    pltpu.sync_copy(src_hbm.at[c, s], src_v)
    pltpu.sync_copy(tgt_hbm.at[s], tgt_v)
    pltpu.sync_copy(s_hbm, s_v)
    pltpu.sync_copy(r_hbm, r_v)

    coff = c * N_PAD

    def _scale(j, rows_x, X):
        @pl.loop(0, CHUNK // 16)
        def _grp(k):
            sl = pl.ds(k * 16, 16)
            idx16 = src_v[j, sl] - coff
            typ16 = typb[X, sl]
            sv = plsc.load_gather(s_v, [idx16])
            rv = plsc.load_gather(r_v, [typ16])
            a16 = 1.0 / (1.0 + jnp.exp(-(sv + rv)))
            base = k * 16
            for l in range(16):
                a = lax.broadcast_in_dim(a16[l], (16,), ())
                for h in range(DH // 16):
                    fsl = pl.ds(h * 16, 16)
                    rows_x[base + l, fsl] = rows_x[base + l, fsl] * a

    HB = CHUNK // 2

    def _gth(j, X):
        pltpu.async_copy(xt_hbm.at[src_v.at[j, pl.ds(0, HB)]],
                         bufs[X].at[pl.ds(0, HB)], gsem.at[X])
        pltpu.async_copy(xt_hbm.at[src_v.at[j, pl.ds(HB, HB)]],
                         bufs[X].at[pl.ds(HB, HB)], gsemb.at[X])

    def _gth_wait(j, X):
        pltpu.make_async_copy(xt_hbm.at[src_v.at[j, pl.ds(0, HB)]],
                              bufs[X].at[pl.ds(0, HB)], gsem.at[X]).wait()
        pltpu.make_async_copy(xt_hbm.at[src_v.at[j, pl.ds(HB, HB)]],
                              bufs[X].at[pl.ds(HB, HB)], gsemb.at[X]).wait()

    def _typ_pre(j, X):
        pltpu.async_copy(typ_hbm.at[s, j], typb.at[X], tsem.at[X])

    for j in range(2):
        _gth(j, j)
        _typ_pre(j, j)

    @pl.loop(0, CHUNKS_PER_W, step=4)
    def _t(t):
        for i in range(4):
            j = t + i
            X = i
            Z = (i + 2) % 4
            _gth_wait(j, X)
            pltpu.make_async_copy(
                typ_hbm.at[s, j], typb.at[X], tsem.at[X]).wait()
            if i < 2:
                @pl.when(t >= 4)
                def _():
                    pltpu.make_async_copy(
                        bufs[Z], acc.at[tgt_v.at[j - 2]], ssem.at[Z]).wait()
                _gth(j + 2, Z)
                _typ_pre(j + 2, Z)
            else:
                pltpu.make_async_copy(
                    bufs[Z], acc.at[tgt_v.at[j - 2]], ssem.at[Z]).wait()

                @pl.when(j + 2 < CHUNKS_PER_W)
                def _():
                    _gth(j + 2, Z)
                    _typ_pre(j + 2, Z)
            _scale(j, bufs[X], X)
            pltpu.async_copy(bufs[X], acc.at[tgt_v.at[j]], ssem.at[X],
                             add=True)

    for i in (2, 3):
        pltpu.make_async_copy(
            bufs[i], acc.at[tgt_v.at[CHUNKS_PER_W - 4 + i]],
            ssem.at[i]).wait()

    plsc.subcore_barrier()
    for b in range(5):
        rbase = s * 625 + b * 125
        pltpu.sync_copy(acc.at[pl.ds(rbase, 125)], rows0.at[pl.ds(0, 125)])

        @pl.loop(0, 125)
        def _relu(i):
            for h in range(DH // 16):
                fsl = pl.ds(h * 16, 16)
                rows0[i, fsl] = jnp.maximum(rows0[i, fsl], 0.0)

        pltpu.sync_copy(rows0.at[pl.ds(0, 125)],
                        out_hbm.at[pl.ds(rbase, 125), pl.ds(c * DH, DH)])


@jax.jit
def _run(x, edge_index, edge_type, rel_emb, W_lin, W_attn):
    src = edge_index[0].astype(jnp.int32)
    tgt = edge_index[1].astype(jnp.int32)
    typ = edge_type.astype(jnp.int32)

    pad = E_PAD - N_EDGES
    src = jnp.concatenate([src, jnp.full((pad,), N_NODES, jnp.int32)])
    tgt = jnp.concatenate([tgt, jnp.zeros((pad,), jnp.int32)])
    typ = jnp.concatenate([typ, jnp.zeros((pad,), jnp.int32)])
    src = src.reshape(NS, CHUNKS_PER_W, CHUNK)
    tgt = tgt.reshape(NS, CHUNKS_PER_W, CHUNK)
    typ = typ.reshape(NS, CHUNKS_PER_W, CHUNK)
    src_off = src[None] + (jnp.arange(NC, dtype=jnp.int32) * N_PAD)[
        :, None, None, None]

    rel_pad = jnp.concatenate(
        [rel_emb, jnp.zeros((64 - N_REL, D), jnp.float32)], axis=0)

    xt_split, s_pad, r_pad = pl.pallas_call(
        _tc_prep,
        out_shape=[
            jax.ShapeDtypeStruct((NC * N_PAD, DH), jnp.float32),
            jax.ShapeDtypeStruct((N_PAD, 1), jnp.float32),
            jax.ShapeDtypeStruct((64, 1), jnp.float32),
        ],
    )(x, W_lin, W_attn, rel_pad)

    s1 = s_pad.reshape(N_PAD)
    r1 = r_pad.reshape(64)

    mesh = plsc.VectorSubcoreMesh(
        core_axis_name="c", subcore_axis_name="s",
        num_cores=NC, num_subcores=NS)
    sc_call = pl.kernel(
        _sc_edges,
        out_type=jax.ShapeDtypeStruct((N_NODES, D), jnp.float32),
        mesh=mesh,
        compiler_params=pltpu.CompilerParams(
            needs_layout_passes=False, use_tc_tiling_on_sc=False),
        scratch_types=[
            pltpu.VMEM((CHUNKS_PER_W, CHUNK), jnp.int32),
            pltpu.VMEM((CHUNKS_PER_W, CHUNK), jnp.int32),
            pltpu.VMEM((N_PAD,), jnp.float32),
            pltpu.VMEM((64,), jnp.float32),
            pltpu.VMEM((4, CHUNK), jnp.int32),
            pltpu.VMEM((CHUNK, DH), jnp.float32),
            pltpu.VMEM((CHUNK, DH), jnp.float32),
            pltpu.VMEM((CHUNK, DH), jnp.float32),
            pltpu.VMEM((CHUNK, DH), jnp.float32),
            pltpu.VMEM_SHARED((ACC_ROWS, DH), jnp.float32),
            pltpu.SemaphoreType.DMA((4,)),
            pltpu.SemaphoreType.DMA((4,)),
            pltpu.SemaphoreType.DMA((4,)),
            pltpu.SemaphoreType.DMA((4,)),
        ],
    )
    out = sc_call(xt_split, s1, r1, src_off, tgt, typ)
    return out


def kernel(x, edge_index, edge_type, rel_emb, W_lin, W_attn):
    return _run(x, edge_index, edge_type, rel_emb, W_lin, W_attn)
